# Initial kernel scaffold; baseline (speedup 1.0000x reference)
import functools

import jax
import jax.numpy as jnp
from jax import lax
from jax.experimental import pallas as pl
from jax.experimental.pallas import tpu as pltpu
from jax.experimental.pallas import tpu_sc as plsc

N = 10000
E = 320000
D = 128
H = 128
C = 10
G = 64

NC = 2
NS = 16
NW = NC * NS
LANES = 16

CH = 128
NCHUNK = 79
EPT = CH * NCHUNK
E_PAD = EPT * NW
N_PAD = 10240
RPT = N_PAD // NS

_mesh = plsc.VectorSubcoreMesh(
    core_axis_name="c", subcore_axis_name="s", num_cores=NC, num_subcores=NS
)


@functools.partial(
    pl.kernel,
    out_type=jax.ShapeDtypeStruct((NC, N_PAD), jnp.float32),
    mesh=_mesh,
    scratch_types=[
        pltpu.VMEM((NCHUNK, CH), jnp.int32),
        pltpu.VMEM((CH,), jnp.float32),
        pltpu.VMEM((RPT,), jnp.float32),
        pltpu.VMEM_SHARED((N_PAD,), jnp.float32),
    ],
)
def _sc_degree(dst_hbm, out_hbm, dstv, ones_v, stage_v, acc_sh):
    c = lax.axis_index("c")
    s = lax.axis_index("s")
    wid = c * NS + s
    pltpu.sync_copy(dst_hbm.at[pl.ds(wid * NCHUNK, NCHUNK)], dstv)
    for j in range(CH // LANES):
        ones_v[pl.ds(j * LANES, LANES)] = jnp.ones((LANES,), jnp.float32)
    zero = jnp.zeros((LANES,), jnp.float32)

    def zfill(i, carry):
        stage_v[pl.ds(i * LANES, LANES)] = zero
        return carry

    lax.fori_loop(0, RPT // LANES, zfill, 0)
    pltpu.sync_copy(stage_v, acc_sh.at[pl.ds(s * RPT, RPT)])
    plsc.subcore_barrier()

    def body(g, carry):
        pltpu.sync_copy(ones_v, acc_sh.at[dstv.at[g]], add=True)
        return carry

    lax.fori_loop(0, NCHUNK, body, 0)
    plsc.subcore_barrier()
    pltpu.sync_copy(acc_sh.at[pl.ds(s * RPT, RPT)], stage_v)
    pltpu.sync_copy(stage_v, out_hbm.at[c, pl.ds(s * RPT, RPT)])


@functools.partial(
    pl.kernel,
    out_type=jax.ShapeDtypeStruct((NC, N_PAD, H), jnp.float32),
    mesh=_mesh,
    scratch_types=[
        pltpu.VMEM((NCHUNK, CH), jnp.int32),
        pltpu.VMEM((NCHUNK, CH), jnp.int32),
        pltpu.VMEM((2, CH, H), jnp.float32),
        pltpu.VMEM_SHARED((N_PAD, H), jnp.float32),
    ],
)
def _sc_aggregate(y_hbm, src_hbm, dst_hbm, out_hbm, srcv, dstv, rows, acc_sh):
    c = lax.axis_index("c")
    s = lax.axis_index("s")
    wid = c * NS + s
    pltpu.sync_copy(src_hbm.at[pl.ds(wid * NCHUNK, NCHUNK)], srcv)
    pltpu.sync_copy(dst_hbm.at[pl.ds(wid * NCHUNK, NCHUNK)], dstv)

    zero = jnp.zeros((LANES,), jnp.float32)

    def zfill(i, carry):
        for j in range(H // LANES):
            rows[0, i, pl.ds(j * LANES, LANES)] = zero
        return carry

    lax.fori_loop(0, CH, zfill, 0)
    for k in range(RPT // CH):
        pltpu.sync_copy(rows.at[0], acc_sh.at[pl.ds(s * RPT + k * CH, CH)])
    plsc.subcore_barrier()

    def body(g, carry):
        pltpu.sync_copy(y_hbm.at[srcv.at[g]], rows.at[0])
        pltpu.sync_copy(rows.at[0], acc_sh.at[dstv.at[g]], add=True)
        return carry

    lax.fori_loop(0, NCHUNK, body, 0)
    plsc.subcore_barrier()
    for k in range(RPT // CH):
        pltpu.sync_copy(acc_sh.at[pl.ds(s * RPT + k * CH, CH)], rows.at[1])
        pltpu.sync_copy(rows.at[1], out_hbm.at[c, pl.ds(s * RPT + k * CH, CH)])


def _tc_scale_body(x_ref, w_ref, degp_ref, y_ref, dinv_ref):
    deg = degp_ref[0] + degp_ref[1] + 1.0
    dinv = lax.rsqrt(deg)
    xw = jnp.dot(x_ref[...], w_ref[...], preferred_element_type=jnp.float32)
    y_ref[...] = xw * dinv
    dinv_ref[...] = dinv


def _tc_mid_body(p_ref, y1_ref, dinv_ref, w2_ref, b1_ref, y2_ref):
    dinv = dinv_ref[...]
    h1 = (p_ref[0] + p_ref[1] + y1_ref[...]) * dinv + b1_ref[...]
    h1 = jnp.maximum(h1, 0.0)
    y2_ref[...] = (
        jnp.dot(h1, w2_ref[...], preferred_element_type=jnp.float32) * dinv
    )


def _tc_head_body(p_ref, y2_ref, dinv_ref, b2_ref, batch_ref,
                  wl1_ref, bl1_ref, wl2_ref, bl2_ref, out_ref):
    h2 = (p_ref[0] + p_ref[1] + y2_ref[...]) * dinv_ref[...] + b2_ref[...]
    h2 = jnp.maximum(h2, 0.0)
    ids = batch_ref[...]
    gidx = lax.broadcasted_iota(jnp.int32, (G, N_PAD), 0)
    oh = (gidx == ids).astype(jnp.float32)
    seg = jnp.dot(oh, h2, preferred_element_type=jnp.float32)
    cnt = jnp.sum(oh, axis=1, keepdims=True)
    pooled = seg / jnp.maximum(cnt, 1.0)
    z = jnp.dot(pooled, wl1_ref[...], preferred_element_type=jnp.float32)
    z = jnp.maximum(z + bl1_ref[...], 0.0)
    logits = jnp.dot(z, wl2_ref[...], preferred_element_type=jnp.float32)
    logits = logits + bl2_ref[...]
    m = jnp.max(logits, axis=1, keepdims=True)
    lse = jnp.log(jnp.sum(jnp.exp(logits - m), axis=1, keepdims=True)) + m
    out_ref[...] = logits - lse


def kernel(x, edge_index, batch, W1, b1, W2, b2, Wl1, bl1, Wl2, bl2):
    f32 = jnp.float32
    src = edge_index[0]
    dst = edge_index[1]

    n_pad_e = E_PAD - E
    dummy = N + (jnp.arange(n_pad_e, dtype=jnp.int32) % (N_PAD - N))
    src_p = jnp.concatenate([src, dummy]).reshape(NW * NCHUNK, CH)
    dst_p = jnp.concatenate([dst, dummy]).reshape(NW * NCHUNK, CH)

    x_p = jnp.pad(x, ((0, N_PAD - N), (0, 0)))
    batch_p = jnp.concatenate(
        [batch, jnp.full((N_PAD - N,), G, jnp.int32)]
    ).reshape(1, N_PAD)

    degp = _sc_degree(dst_p).reshape(NC, N_PAD, 1)

    y1, dinv = pl.pallas_call(
        _tc_scale_body,
        out_shape=[
            jax.ShapeDtypeStruct((N_PAD, H), f32),
            jax.ShapeDtypeStruct((N_PAD, 1), f32),
        ],
    )(x_p, W1, degp)

    p1 = _sc_aggregate(y1, src_p, dst_p)

    y2 = pl.pallas_call(
        _tc_mid_body,
        out_shape=jax.ShapeDtypeStruct((N_PAD, H), f32),
    )(p1, y1, dinv, W2, b1.reshape(1, H))

    p2 = _sc_aggregate(y2, src_p, dst_p)

    out = pl.pallas_call(
        _tc_head_body,
        out_shape=jax.ShapeDtypeStruct((G, C), f32),
    )(p2, y2, dinv, b2.reshape(1, H), batch_p,
      Wl1, bl1.reshape(1, H), Wl2, bl2.reshape(1, C))
    return out


# speedup vs baseline: 22.9799x; 22.9799x over previous
import functools

import jax
import jax.numpy as jnp
from jax import lax
from jax.experimental import pallas as pl
from jax.experimental.pallas import tpu as pltpu
from jax.experimental.pallas import tpu_sc as plsc

N = 10000
E = 320000
D = 128
H = 128
C = 10
G = 64

NC = 2
NS = 16
NW = NC * NS
LANES = 16

CH = 128
NCHUNK = 80
EPT = CH * NCHUNK
E_PAD = EPT * NW
N_PAD = 10240
RPT = N_PAD // NS


@functools.cache
def _make_sc_degree():
    mesh = plsc.VectorSubcoreMesh(
        core_axis_name="c", subcore_axis_name="s",
        num_cores=NC, num_subcores=NS,
    )
    return pl.kernel(
        _sc_degree,
        out_type=jax.ShapeDtypeStruct((NC, 1, N_PAD), jnp.float32),
        mesh=mesh,
        scratch_types=[
            pltpu.VMEM((NCHUNK, CH), jnp.int32),
            pltpu.VMEM((CH,), jnp.float32),
            pltpu.VMEM((RPT,), jnp.float32),
            pltpu.VMEM_SHARED((N_PAD,), jnp.float32),
        ],
    )


def _sc_degree(dst_hbm, out_hbm, dstv, ones_v, stage_v, acc_sh):
    c = lax.axis_index("c")
    s = lax.axis_index("s")
    wid = c * NS + s
    pltpu.sync_copy(dst_hbm.at[pl.ds(wid * NCHUNK, NCHUNK)], dstv)
    for j in range(CH // LANES):
        ones_v[pl.ds(j * LANES, LANES)] = jnp.ones((LANES,), jnp.float32)
    zero = jnp.zeros((LANES,), jnp.float32)

    def zfill(i, carry):
        stage_v[pl.ds(i * LANES, LANES)] = zero
        return carry

    lax.fori_loop(0, RPT // LANES, zfill, 0)
    pltpu.sync_copy(stage_v, acc_sh.at[pl.ds(s * RPT, RPT)])
    plsc.subcore_barrier()

    def body(g, carry):
        pltpu.sync_copy(ones_v, acc_sh.at[dstv.at[g]], add=True)
        return carry

    lax.fori_loop(0, NCHUNK, body, 0)
    plsc.subcore_barrier()
    pltpu.sync_copy(acc_sh.at[pl.ds(s * RPT, RPT)], stage_v)
    pltpu.sync_copy(stage_v, out_hbm.at[c, 0, pl.ds(s * RPT, RPT)])


@functools.cache
def _make_sc_aggregate():
    mesh = plsc.VectorSubcoreMesh(
        core_axis_name="c", subcore_axis_name="s",
        num_cores=NC, num_subcores=NS,
    )
    return pl.kernel(
        _sc_aggregate,
        out_type=jax.ShapeDtypeStruct((NC, N_PAD, H), jnp.float32),
        mesh=mesh,
        scratch_types=[
            pltpu.VMEM((NCHUNK, CH), jnp.int32),
            pltpu.VMEM((NCHUNK, CH), jnp.int32),
            pltpu.VMEM((1, CH, H), jnp.float32),
            pltpu.VMEM_SHARED((N_PAD, H), jnp.float32),
        ],
    )


def _sc_aggregate(y_hbm, src_hbm, dst_hbm, out_hbm, srcv, dstv, rows, acc_sh):
    c = lax.axis_index("c")
    s = lax.axis_index("s")
    wid = c * NS + s
    pltpu.sync_copy(src_hbm.at[pl.ds(wid * NCHUNK, NCHUNK)], srcv)
    pltpu.sync_copy(dst_hbm.at[pl.ds(wid * NCHUNK, NCHUNK)], dstv)

    zero = jnp.zeros((LANES,), jnp.float32)

    def zfill(i, carry):
        for j in range(H // LANES):
            rows[0, i, pl.ds(j * LANES, LANES)] = zero
        return carry

    lax.fori_loop(0, CH, zfill, 0)
    for k in range(RPT // CH):
        pltpu.sync_copy(rows.at[0], acc_sh.at[pl.ds(s * RPT + k * CH, CH)])
    plsc.subcore_barrier()

    def body(g, carry):
        pltpu.sync_copy(y_hbm.at[srcv.at[g]], rows.at[0])
        pltpu.sync_copy(rows.at[0], acc_sh.at[dstv.at[g]], add=True)
        return carry

    lax.fori_loop(0, NCHUNK, body, 0)
    plsc.subcore_barrier()
    for k in range(RPT // CH):
        pltpu.sync_copy(acc_sh.at[pl.ds(s * RPT + k * CH, CH)], rows.at[0])
        pltpu.sync_copy(rows.at[0], out_hbm.at[c, pl.ds(s * RPT + k * CH, CH)])


def _tc_scale_body(x_ref, w_ref, degp_ref, y_ref, dinv_ref):
    deg = degp_ref[0] + degp_ref[1] + 1.0
    dinv = lax.rsqrt(deg)
    xw = jnp.dot(x_ref[...], w_ref[...], preferred_element_type=jnp.float32)
    y_ref[...] = xw * dinv
    dinv_ref[...] = dinv


def _tc_mid_body(p_ref, y1_ref, dinv_ref, w2_ref, b1_ref, y2_ref):
    dinv = dinv_ref[...]
    h1 = (p_ref[0] + p_ref[1] + y1_ref[...]) * dinv + b1_ref[...]
    h1 = jnp.maximum(h1, 0.0)
    y2_ref[...] = (
        jnp.dot(h1, w2_ref[...], preferred_element_type=jnp.float32) * dinv
    )


def _tc_head_body(p_ref, y2_ref, dinv_ref, b2_ref, batch_ref,
                  wl1_ref, bl1_ref, wl2_ref, bl2_ref, out_ref):
    h2 = (p_ref[0] + p_ref[1] + y2_ref[...]) * dinv_ref[...] + b2_ref[...]
    h2 = jnp.maximum(h2, 0.0)
    ids = batch_ref[...]
    gidx = lax.broadcasted_iota(jnp.int32, (G, N_PAD), 0)
    oh = (gidx == ids).astype(jnp.float32)
    seg = jnp.dot(oh, h2, preferred_element_type=jnp.float32)
    cnt = jnp.sum(oh, axis=1, keepdims=True)
    pooled = seg / jnp.maximum(cnt, 1.0)
    z = jnp.dot(pooled, wl1_ref[...], preferred_element_type=jnp.float32)
    z = jnp.maximum(z + bl1_ref[...], 0.0)
    logits = jnp.dot(z, wl2_ref[...], preferred_element_type=jnp.float32)
    logits = logits + bl2_ref[...]
    m = jnp.max(logits, axis=1, keepdims=True)
    lse = jnp.log(jnp.sum(jnp.exp(logits - m), axis=1, keepdims=True)) + m
    out_ref[...] = logits - lse


def kernel(x, edge_index, batch, W1, b1, W2, b2, Wl1, bl1, Wl2, bl2):
    f32 = jnp.float32
    src = edge_index[0]
    dst = edge_index[1]

    n_pad_e = E_PAD - E
    dummy = N + (jnp.arange(n_pad_e, dtype=jnp.int32) % (N_PAD - N))
    src_p = jnp.concatenate([src, dummy]).reshape(NW * NCHUNK, CH)
    dst_p = jnp.concatenate([dst, dummy]).reshape(NW * NCHUNK, CH)

    x_p = jnp.pad(x, ((0, N_PAD - N), (0, 0)))
    batch_p = jnp.concatenate(
        [batch, jnp.full((N_PAD - N,), G, jnp.int32)]
    ).reshape(1, N_PAD)

    degp = _make_sc_degree()(dst_p).reshape(NC, N_PAD, 1)

    y1, dinv = pl.pallas_call(
        _tc_scale_body,
        out_shape=[
            jax.ShapeDtypeStruct((N_PAD, H), f32),
            jax.ShapeDtypeStruct((N_PAD, 1), f32),
        ],
    )(x_p, W1, degp)

    p1 = _make_sc_aggregate()(y1, src_p, dst_p)

    y2 = pl.pallas_call(
        _tc_mid_body,
        out_shape=jax.ShapeDtypeStruct((N_PAD, H), f32),
    )(p1, y1, dinv, W2, b1.reshape(1, H))

    p2 = _make_sc_aggregate()(y2, src_p, dst_p)

    out = pl.pallas_call(
        _tc_head_body,
        out_shape=jax.ShapeDtypeStruct((G, C), f32),
    )(p2, y2, dinv, b2.reshape(1, H), batch_p,
      Wl1, bl1.reshape(1, H), Wl2, bl2.reshape(1, C))
    return out


# speedup vs baseline: 33.1040x; 1.4406x over previous
import functools

import jax
import jax.numpy as jnp
from jax import lax
from jax.experimental import pallas as pl
from jax.experimental.pallas import tpu as pltpu
from jax.experimental.pallas import tpu_sc as plsc

N = 10000
E = 320000
D = 128
H = 128
C = 10
G = 64

NC = 2
NS = 16
NW = NC * NS
LANES = 16

CH = 128
NCHUNK = 80
PH = 2
CPP = NCHUNK // PH
EPT = CH * NCHUNK
E_PAD = EPT * NW
N_PAD = 10240
RPT = N_PAD // NS


@functools.cache
def _make_sc_degree():
    mesh = plsc.VectorSubcoreMesh(
        core_axis_name="c", subcore_axis_name="s",
        num_cores=NC, num_subcores=NS,
    )
    return pl.kernel(
        _sc_degree,
        out_type=jax.ShapeDtypeStruct((NC, 1, N_PAD), jnp.float32),
        mesh=mesh,
        scratch_types=[
            pltpu.VMEM((NCHUNK, CH), jnp.int32),
            pltpu.VMEM((CH,), jnp.float32),
            pltpu.VMEM((RPT,), jnp.float32),
            pltpu.VMEM_SHARED((N_PAD,), jnp.float32),
        ],
    )


def _sc_degree(dst_hbm, out_hbm, dstv, ones_v, stage_v, acc_sh):
    c = lax.axis_index("c")
    s = lax.axis_index("s")
    wid = c * NS + s
    pltpu.sync_copy(dst_hbm.at[pl.ds(wid * NCHUNK, NCHUNK)], dstv)
    for j in range(CH // LANES):
        ones_v[pl.ds(j * LANES, LANES)] = jnp.ones((LANES,), jnp.float32)
    zero = jnp.zeros((LANES,), jnp.float32)

    def zfill(i, carry):
        stage_v[pl.ds(i * LANES, LANES)] = zero
        return carry

    lax.fori_loop(0, RPT // LANES, zfill, 0)
    pltpu.sync_copy(stage_v, acc_sh.at[pl.ds(s * RPT, RPT)])
    plsc.subcore_barrier()

    def body(g, carry):
        pltpu.sync_copy(ones_v, acc_sh.at[dstv.at[g]], add=True)
        return carry

    lax.fori_loop(0, NCHUNK, body, 0)
    plsc.subcore_barrier()
    pltpu.sync_copy(acc_sh.at[pl.ds(s * RPT, RPT)], stage_v)
    pltpu.sync_copy(stage_v, out_hbm.at[c, 0, pl.ds(s * RPT, RPT)])


@functools.cache
def _make_sc_aggregate():
    mesh = plsc.VectorSubcoreMesh(
        core_axis_name="c", subcore_axis_name="s",
        num_cores=NC, num_subcores=NS,
    )
    return pl.kernel(
        _sc_aggregate,
        out_type=jax.ShapeDtypeStruct((NC, N_PAD, H), jnp.float32),
        mesh=mesh,
        scratch_types=[
            pltpu.VMEM((CPP, CH), jnp.int32),
            pltpu.VMEM((CPP, CH), jnp.int32),
            pltpu.VMEM((2, CH, H), jnp.float32),
            pltpu.VMEM_SHARED((N_PAD, H), jnp.float32),
            pltpu.SemaphoreType.DMA,
            pltpu.SemaphoreType.DMA,
        ],
    )


def _sc_aggregate(y_hbm, src_hbm, dst_hbm, out_hbm, srcv, dstv, rows, acc_sh,
                  sem0, sem1):
    c = lax.axis_index("c")
    s = lax.axis_index("s")
    wid = c * NS + s

    zero = jnp.zeros((LANES,), jnp.float32)

    def zfill(i, carry):
        for j in range(H // LANES):
            rows[0, i, pl.ds(j * LANES, LANES)] = zero
        return carry

    lax.fori_loop(0, CH, zfill, 0)
    for k in range(RPT // CH):
        pltpu.sync_copy(rows.at[0], acc_sh.at[pl.ds(s * RPT + k * CH, CH)])
    plsc.subcore_barrier()

    def gather(g, buf, sem):
        return pltpu.async_copy(y_hbm.at[srcv.at[g]], rows.at[buf], sem)

    for p in range(PH):
        base = wid * NCHUNK + p * CPP
        pltpu.sync_copy(src_hbm.at[pl.ds(base, CPP)], srcv)
        pltpu.sync_copy(dst_hbm.at[pl.ds(base, CPP)], dstv)
        gather(0, 0, sem0)

        def body(j, carry):
            g0 = 2 * j
            g1 = g0 + 1
            gather(g1, 1, sem1)
            pltpu.make_async_copy(
                y_hbm.at[srcv.at[g0]], rows.at[0], sem0).wait()
            pltpu.sync_copy(rows.at[0], acc_sh.at[dstv.at[g0]], add=True)

            @pl.when(g0 + 2 < CPP)
            def _():
                gather(g0 + 2, 0, sem0)

            pltpu.make_async_copy(
                y_hbm.at[srcv.at[g1]], rows.at[1], sem1).wait()
            pltpu.sync_copy(rows.at[1], acc_sh.at[dstv.at[g1]], add=True)
            return carry

        lax.fori_loop(0, CPP // 2, body, 0)
    plsc.subcore_barrier()
    for k in range(RPT // CH):
        pltpu.sync_copy(acc_sh.at[pl.ds(s * RPT + k * CH, CH)], rows.at[0])
        pltpu.sync_copy(rows.at[0], out_hbm.at[c, pl.ds(s * RPT + k * CH, CH)])


def _tc_scale_body(x_ref, w_ref, degp_ref, y_ref, dinv_ref):
    deg = degp_ref[0] + degp_ref[1] + 1.0
    dinv = lax.rsqrt(deg)
    xw = jnp.dot(x_ref[...], w_ref[...], preferred_element_type=jnp.float32)
    y_ref[...] = xw * dinv
    dinv_ref[...] = dinv


def _tc_mid_body(p_ref, y1_ref, dinv_ref, w2_ref, b1_ref, y2_ref):
    dinv = dinv_ref[...]
    h1 = (p_ref[0] + p_ref[1] + y1_ref[...]) * dinv + b1_ref[...]
    h1 = jnp.maximum(h1, 0.0)
    y2_ref[...] = (
        jnp.dot(h1, w2_ref[...], preferred_element_type=jnp.float32) * dinv
    )


def _tc_head_body(p_ref, y2_ref, dinv_ref, b2_ref, batch_ref,
                  wl1_ref, bl1_ref, wl2_ref, bl2_ref, out_ref):
    h2 = (p_ref[0] + p_ref[1] + y2_ref[...]) * dinv_ref[...] + b2_ref[...]
    h2 = jnp.maximum(h2, 0.0)
    ids = batch_ref[...]
    gidx = lax.broadcasted_iota(jnp.int32, (G, N_PAD), 0)
    oh = (gidx == ids).astype(jnp.float32)
    seg = jnp.dot(oh, h2, preferred_element_type=jnp.float32)
    cnt = jnp.sum(oh, axis=1, keepdims=True)
    pooled = seg / jnp.maximum(cnt, 1.0)
    z = jnp.dot(pooled, wl1_ref[...], preferred_element_type=jnp.float32)
    z = jnp.maximum(z + bl1_ref[...], 0.0)
    logits = jnp.dot(z, wl2_ref[...], preferred_element_type=jnp.float32)
    logits = logits + bl2_ref[...]
    m = jnp.max(logits, axis=1, keepdims=True)
    lse = jnp.log(jnp.sum(jnp.exp(logits - m), axis=1, keepdims=True)) + m
    out_ref[...] = logits - lse


def kernel(x, edge_index, batch, W1, b1, W2, b2, Wl1, bl1, Wl2, bl2):
    f32 = jnp.float32
    src = edge_index[0]
    dst = edge_index[1]

    n_pad_e = E_PAD - E
    dummy = N + (jnp.arange(n_pad_e, dtype=jnp.int32) % (N_PAD - N))
    src_p = jnp.concatenate([src, dummy]).reshape(NW * NCHUNK, CH)
    dst_p = jnp.concatenate([dst, dummy]).reshape(NW * NCHUNK, CH)

    x_p = jnp.pad(x, ((0, N_PAD - N), (0, 0)))
    batch_p = jnp.concatenate(
        [batch, jnp.full((N_PAD - N,), G, jnp.int32)]
    ).reshape(1, N_PAD)

    degp = _make_sc_degree()(dst_p).reshape(NC, N_PAD, 1)

    y1, dinv = pl.pallas_call(
        _tc_scale_body,
        out_shape=[
            jax.ShapeDtypeStruct((N_PAD, H), f32),
            jax.ShapeDtypeStruct((N_PAD, 1), f32),
        ],
    )(x_p, W1, degp)

    p1 = _make_sc_aggregate()(y1, src_p, dst_p)

    y2 = pl.pallas_call(
        _tc_mid_body,
        out_shape=jax.ShapeDtypeStruct((N_PAD, H), f32),
    )(p1, y1, dinv, W2, b1.reshape(1, H))

    p2 = _make_sc_aggregate()(y2, src_p, dst_p)

    out = pl.pallas_call(
        _tc_head_body,
        out_shape=jax.ShapeDtypeStruct((G, C), f32),
    )(p2, y2, dinv, b2.reshape(1, H), batch_p,
      Wl1, bl1.reshape(1, H), Wl2, bl2.reshape(1, C))
    return out


# speedup vs baseline: 33.5696x; 1.0141x over previous
import functools

import jax
import jax.numpy as jnp
from jax import lax
from jax.experimental import pallas as pl
from jax.experimental.pallas import tpu as pltpu
from jax.experimental.pallas import tpu_sc as plsc

N = 10000
E = 320000
D = 128
H = 128
C = 10
G = 64

NC = 2
NS = 16
NW = NC * NS
LANES = 16

CH = 64
NCHUNK = 160
PH = 4
NBUF = 4
CPP = NCHUNK // PH
EPT = CH * NCHUNK
E_PAD = EPT * NW
N_PAD = 10240
RPT = N_PAD // NS


@functools.cache
def _make_sc_degree():
    mesh = plsc.VectorSubcoreMesh(
        core_axis_name="c", subcore_axis_name="s",
        num_cores=NC, num_subcores=NS,
    )
    return pl.kernel(
        _sc_degree,
        out_type=jax.ShapeDtypeStruct((NC, 1, N_PAD), jnp.float32),
        mesh=mesh,
        scratch_types=[
            pltpu.VMEM((NCHUNK, CH), jnp.int32),
            pltpu.VMEM((CH,), jnp.float32),
            pltpu.VMEM((RPT,), jnp.float32),
            pltpu.VMEM_SHARED((N_PAD,), jnp.float32),
        ],
    )


def _sc_degree(dst_hbm, out_hbm, dstv, ones_v, stage_v, acc_sh):
    c = lax.axis_index("c")
    s = lax.axis_index("s")
    wid = c * NS + s
    pltpu.sync_copy(dst_hbm.at[pl.ds(wid * NCHUNK, NCHUNK)], dstv)
    for j in range(CH // LANES):
        ones_v[pl.ds(j * LANES, LANES)] = jnp.ones((LANES,), jnp.float32)
    zero = jnp.zeros((LANES,), jnp.float32)

    def zfill(i, carry):
        stage_v[pl.ds(i * LANES, LANES)] = zero
        return carry

    lax.fori_loop(0, RPT // LANES, zfill, 0)
    pltpu.sync_copy(stage_v, acc_sh.at[pl.ds(s * RPT, RPT)])
    plsc.subcore_barrier()

    def body(g, carry):
        pltpu.sync_copy(ones_v, acc_sh.at[dstv.at[g]], add=True)
        return carry

    lax.fori_loop(0, NCHUNK, body, 0)
    plsc.subcore_barrier()
    pltpu.sync_copy(acc_sh.at[pl.ds(s * RPT, RPT)], stage_v)
    pltpu.sync_copy(stage_v, out_hbm.at[c, 0, pl.ds(s * RPT, RPT)])


@functools.cache
def _make_sc_aggregate():
    mesh = plsc.VectorSubcoreMesh(
        core_axis_name="c", subcore_axis_name="s",
        num_cores=NC, num_subcores=NS,
    )
    return pl.kernel(
        _sc_aggregate,
        out_type=jax.ShapeDtypeStruct((NC, N_PAD, H), jnp.float32),
        mesh=mesh,
        scratch_types=[
            pltpu.VMEM((CPP, CH), jnp.int32),
            pltpu.VMEM((CPP, CH), jnp.int32),
            pltpu.VMEM((NBUF, CH, H), jnp.float32),
            pltpu.VMEM_SHARED((N_PAD, H), jnp.float32),
            [pltpu.SemaphoreType.DMA] * NBUF,
        ],
    )


def _sc_aggregate(y_hbm, src_hbm, dst_hbm, out_hbm, srcv, dstv, rows, acc_sh,
                  sems):
    c = lax.axis_index("c")
    s = lax.axis_index("s")
    wid = c * NS + s

    zero = jnp.zeros((LANES,), jnp.float32)

    def zfill(i, carry):
        for j in range(H // LANES):
            rows[0, i, pl.ds(j * LANES, LANES)] = zero
        return carry

    lax.fori_loop(0, CH, zfill, 0)
    for k in range(RPT // CH):
        pltpu.sync_copy(rows.at[0], acc_sh.at[pl.ds(s * RPT + k * CH, CH)])
    plsc.subcore_barrier()

    def gather(g, buf):
        return pltpu.async_copy(y_hbm.at[srcv.at[g]], rows.at[buf], sems[buf])

    def gwait(g, buf):
        pltpu.make_async_copy(
            y_hbm.at[srcv.at[g]], rows.at[buf], sems[buf]).wait()

    for p in range(PH):
        base = wid * NCHUNK + p * CPP
        pltpu.sync_copy(src_hbm.at[pl.ds(base, CPP)], srcv)
        pltpu.sync_copy(dst_hbm.at[pl.ds(base, CPP)], dstv)
        for b in range(NBUF - 1):
            gather(b, b)

        def body(j, carry):
            g0 = NBUF * j
            gather(g0 + NBUF - 1, NBUF - 1)
            for b in range(NBUF):
                g = g0 + b
                gwait(g, b)
                pltpu.sync_copy(rows.at[b], acc_sh.at[dstv.at[g]], add=True)
                if b < NBUF - 1:
                    @pl.when(g + NBUF < CPP)
                    def _():
                        gather(g + NBUF, b)
            return carry

        lax.fori_loop(0, CPP // NBUF, body, 0)
    plsc.subcore_barrier()
    for k in range(RPT // CH):
        pltpu.sync_copy(acc_sh.at[pl.ds(s * RPT + k * CH, CH)], rows.at[0])
        pltpu.sync_copy(rows.at[0], out_hbm.at[c, pl.ds(s * RPT + k * CH, CH)])


def _tc_scale_body(x_ref, w_ref, degp_ref, y_ref, dinv_ref):
    deg = degp_ref[0] + degp_ref[1] + 1.0
    dinv = lax.rsqrt(deg)
    xw = jnp.dot(x_ref[...], w_ref[...], preferred_element_type=jnp.float32)
    y_ref[...] = xw * dinv
    dinv_ref[...] = dinv


def _tc_mid_body(p_ref, y1_ref, dinv_ref, w2_ref, b1_ref, y2_ref):
    dinv = dinv_ref[...]
    h1 = (p_ref[0] + p_ref[1] + y1_ref[...]) * dinv + b1_ref[...]
    h1 = jnp.maximum(h1, 0.0)
    y2_ref[...] = (
        jnp.dot(h1, w2_ref[...], preferred_element_type=jnp.float32) * dinv
    )


def _tc_head_body(p_ref, y2_ref, dinv_ref, b2_ref, batch_ref,
                  wl1_ref, bl1_ref, wl2_ref, bl2_ref, out_ref):
    h2 = (p_ref[0] + p_ref[1] + y2_ref[...]) * dinv_ref[...] + b2_ref[...]
    h2 = jnp.maximum(h2, 0.0)
    ids = batch_ref[...]
    gidx = lax.broadcasted_iota(jnp.int32, (G, N_PAD), 0)
    oh = (gidx == ids).astype(jnp.float32)
    seg = jnp.dot(oh, h2, preferred_element_type=jnp.float32)
    cnt = jnp.sum(oh, axis=1, keepdims=True)
    pooled = seg / jnp.maximum(cnt, 1.0)
    z = jnp.dot(pooled, wl1_ref[...], preferred_element_type=jnp.float32)
    z = jnp.maximum(z + bl1_ref[...], 0.0)
    logits = jnp.dot(z, wl2_ref[...], preferred_element_type=jnp.float32)
    logits = logits + bl2_ref[...]
    m = jnp.max(logits, axis=1, keepdims=True)
    lse = jnp.log(jnp.sum(jnp.exp(logits - m), axis=1, keepdims=True)) + m
    out_ref[...] = logits - lse


def kernel(x, edge_index, batch, W1, b1, W2, b2, Wl1, bl1, Wl2, bl2):
    f32 = jnp.float32
    src = edge_index[0]
    dst = edge_index[1]

    n_pad_e = E_PAD - E
    dummy = N + (jnp.arange(n_pad_e, dtype=jnp.int32) % (N_PAD - N))
    src_p = jnp.concatenate([src, dummy]).reshape(NW * NCHUNK, CH)
    dst_p = jnp.concatenate([dst, dummy]).reshape(NW * NCHUNK, CH)

    x_p = jnp.pad(x, ((0, N_PAD - N), (0, 0)))
    batch_p = jnp.concatenate(
        [batch, jnp.full((N_PAD - N,), G, jnp.int32)]
    ).reshape(1, N_PAD)

    degp = _make_sc_degree()(dst_p).reshape(NC, N_PAD, 1)

    y1, dinv = pl.pallas_call(
        _tc_scale_body,
        out_shape=[
            jax.ShapeDtypeStruct((N_PAD, H), f32),
            jax.ShapeDtypeStruct((N_PAD, 1), f32),
        ],
    )(x_p, W1, degp)

    p1 = _make_sc_aggregate()(y1, src_p, dst_p)

    y2 = pl.pallas_call(
        _tc_mid_body,
        out_shape=jax.ShapeDtypeStruct((N_PAD, H), f32),
    )(p1, y1, dinv, W2, b1.reshape(1, H))

    p2 = _make_sc_aggregate()(y2, src_p, dst_p)

    out = pl.pallas_call(
        _tc_head_body,
        out_shape=jax.ShapeDtypeStruct((G, C), f32),
    )(p2, y2, dinv, b2.reshape(1, H), batch_p,
      Wl1, bl1.reshape(1, H), Wl2, bl2.reshape(1, C))
    return out


# speedup vs baseline: 34.1735x; 1.0180x over previous
import functools

import jax
import jax.numpy as jnp
from jax import lax
from jax.experimental import pallas as pl
from jax.experimental.pallas import tpu as pltpu
from jax.experimental.pallas import tpu_sc as plsc

N = 10000
E = 320000
D = 128
H = 128
C = 10
G = 64

NC = 2
NS = 16
NW = NC * NS
LANES = 16

CH = 64
NCHUNK = 160
PH = 4
NBUF = 4
CPP = NCHUNK // PH
EPT = CH * NCHUNK
E_PAD = EPT * NW
N_PAD = 10240
RPT = N_PAD // NS


@functools.cache
def _make_sc_degree():
    mesh = plsc.VectorSubcoreMesh(
        core_axis_name="c", subcore_axis_name="s",
        num_cores=NC, num_subcores=NS,
    )
    return pl.kernel(
        _sc_degree,
        out_type=jax.ShapeDtypeStruct((NC, 1, N_PAD), jnp.float32),
        mesh=mesh,
        scratch_types=[
            pltpu.VMEM((NCHUNK, CH), jnp.int32),
            pltpu.VMEM((CH,), jnp.float32),
            pltpu.VMEM((RPT,), jnp.float32),
            pltpu.VMEM_SHARED((N_PAD,), jnp.float32),
        ],
    )


def _sc_degree(dst_hbm, out_hbm, dstv, ones_v, stage_v, acc_sh):
    c = lax.axis_index("c")
    s = lax.axis_index("s")
    wid = c * NS + s
    pltpu.sync_copy(dst_hbm.at[pl.ds(wid * NCHUNK, NCHUNK)], dstv)
    for j in range(CH // LANES):
        ones_v[pl.ds(j * LANES, LANES)] = jnp.ones((LANES,), jnp.float32)
    zero = jnp.zeros((LANES,), jnp.float32)

    def zfill(i, carry):
        stage_v[pl.ds(i * LANES, LANES)] = zero
        return carry

    lax.fori_loop(0, RPT // LANES, zfill, 0)
    pltpu.sync_copy(stage_v, acc_sh.at[pl.ds(s * RPT, RPT)])
    plsc.subcore_barrier()

    def body(g, carry):
        pltpu.sync_copy(ones_v, acc_sh.at[dstv.at[g]], add=True)
        return carry

    lax.fori_loop(0, NCHUNK, body, 0)
    plsc.subcore_barrier()
    pltpu.sync_copy(acc_sh.at[pl.ds(s * RPT, RPT)], stage_v)
    pltpu.sync_copy(stage_v, out_hbm.at[c, 0, pl.ds(s * RPT, RPT)])


@functools.cache
def _make_sc_aggregate():
    mesh = plsc.VectorSubcoreMesh(
        core_axis_name="c", subcore_axis_name="s",
        num_cores=NC, num_subcores=NS,
    )
    return pl.kernel(
        _sc_aggregate,
        out_type=jax.ShapeDtypeStruct((NC, N_PAD, H), jnp.float32),
        mesh=mesh,
        scratch_types=[
            pltpu.VMEM((CPP, CH), jnp.int32),
            pltpu.VMEM((CPP, CH), jnp.int32),
            pltpu.VMEM((NBUF, CH, H), jnp.float32),
            pltpu.VMEM_SHARED((N_PAD, H), jnp.float32),
            [pltpu.SemaphoreType.DMA] * NBUF,
        ],
    )


def _sc_aggregate(y_hbm, src_hbm, dst_hbm, out_hbm, srcv, dstv, rows, acc_sh,
                  sems):
    c = lax.axis_index("c")
    s = lax.axis_index("s")
    wid = c * NS + s

    zero = jnp.zeros((LANES,), jnp.float32)

    def zfill(i, carry):
        for j in range(H // LANES):
            rows[0, i, pl.ds(j * LANES, LANES)] = zero
        return carry

    lax.fori_loop(0, CH, zfill, 0)
    for k in range(RPT // CH):
        pltpu.sync_copy(rows.at[0], acc_sh.at[pl.ds(s * RPT + k * CH, CH)])
    plsc.subcore_barrier()

    def gather(g, buf):
        return pltpu.async_copy(y_hbm.at[srcv.at[g]], rows.at[buf], sems[buf])

    def gwait(g, buf):
        pltpu.make_async_copy(
            y_hbm.at[srcv.at[g]], rows.at[buf], sems[buf]).wait()

    for p in range(PH):
        base = wid * NCHUNK + p * CPP
        pltpu.sync_copy(src_hbm.at[pl.ds(base, CPP)], srcv)
        pltpu.sync_copy(dst_hbm.at[pl.ds(base, CPP)], dstv)
        for b in range(NBUF - 1):
            gather(b, b)

        def body(j, carry):
            g0 = NBUF * j
            gather(g0 + NBUF - 1, NBUF - 1)
            for b in range(NBUF):
                g = g0 + b
                gwait(g, b)
                pltpu.sync_copy(rows.at[b], acc_sh.at[dstv.at[g]], add=True)
                if b < NBUF - 1:
                    @pl.when(g + NBUF < CPP)
                    def _():
                        gather(g + NBUF, b)
            return carry

        lax.fori_loop(0, CPP // NBUF, body, 0)
    plsc.subcore_barrier()
    nwr = RPT // CH
    for k in range(nwr):
        b = k % NBUF
        if k >= NBUF:
            ko = k - NBUF
            pltpu.make_async_copy(
                rows.at[b],
                out_hbm.at[c, pl.ds(s * RPT + ko * CH, CH)],
                sems[b],
            ).wait()
        pltpu.sync_copy(acc_sh.at[pl.ds(s * RPT + k * CH, CH)], rows.at[b])
        pltpu.async_copy(
            rows.at[b], out_hbm.at[c, pl.ds(s * RPT + k * CH, CH)], sems[b])
    for k in range(max(nwr - NBUF, 0), nwr):
        b = k % NBUF
        pltpu.make_async_copy(
            rows.at[b],
            out_hbm.at[c, pl.ds(s * RPT + k * CH, CH)],
            sems[b],
        ).wait()


def _tc_mm1_body(x_ref, w_ref, xw_ref):
    xw_ref[0:N, :] = jnp.dot(
        x_ref[...], w_ref[...], preferred_element_type=jnp.float32)
    xw_ref[N:N_PAD, :] = jnp.zeros((N_PAD - N, H), jnp.float32)


def _tc_scale_body(xw_ref, degp_ref, y_ref, dinv_ref):
    deg = degp_ref[0] + degp_ref[1] + 1.0
    dinv = lax.rsqrt(deg)
    y_ref[...] = xw_ref[...] * dinv
    dinv_ref[...] = dinv


def _tc_mid_body(p_ref, y1_ref, dinv_ref, w2_ref, b1_ref, y2_ref):
    dinv = dinv_ref[...]
    h1 = (p_ref[0] + p_ref[1] + y1_ref[...]) * dinv + b1_ref[...]
    h1 = jnp.maximum(h1, 0.0)
    y2_ref[...] = (
        jnp.dot(h1, w2_ref[...], preferred_element_type=jnp.float32) * dinv
    )


def _tc_head_body(p_ref, y2_ref, dinv_ref, b2_ref, batch_ref,
                  wl1_ref, bl1_ref, wl2_ref, bl2_ref, out_ref):
    h2 = (p_ref[0] + p_ref[1] + y2_ref[...]) * dinv_ref[...] + b2_ref[...]
    h2 = jnp.maximum(h2, 0.0)
    ids = batch_ref[...]
    gidx = lax.broadcasted_iota(jnp.int32, (G, N_PAD), 0)
    oh = (gidx == ids).astype(jnp.float32)
    seg = jnp.dot(oh, h2, preferred_element_type=jnp.float32)
    cnt = jnp.sum(oh, axis=1, keepdims=True)
    pooled = seg / jnp.maximum(cnt, 1.0)
    z = jnp.dot(pooled, wl1_ref[...], preferred_element_type=jnp.float32)
    z = jnp.maximum(z + bl1_ref[...], 0.0)
    logits = jnp.dot(z, wl2_ref[...], preferred_element_type=jnp.float32)
    logits = logits + bl2_ref[...]
    m = jnp.max(logits, axis=1, keepdims=True)
    lse = jnp.log(jnp.sum(jnp.exp(logits - m), axis=1, keepdims=True)) + m
    out_ref[...] = logits - lse


def kernel(x, edge_index, batch, W1, b1, W2, b2, Wl1, bl1, Wl2, bl2):
    f32 = jnp.float32
    src = edge_index[0]
    dst = edge_index[1]

    n_pad_e = E_PAD - E
    dummy = N + (jnp.arange(n_pad_e, dtype=jnp.int32) % (N_PAD - N))
    src_p = jnp.concatenate([src, dummy]).reshape(NW * NCHUNK, CH)
    dst_p = jnp.concatenate([dst, dummy]).reshape(NW * NCHUNK, CH)

    batch_p = jnp.concatenate(
        [batch, jnp.full((N_PAD - N,), G, jnp.int32)]
    ).reshape(1, N_PAD)

    xw = pl.pallas_call(
        _tc_mm1_body,
        out_shape=jax.ShapeDtypeStruct((N_PAD, H), f32),
    )(x, W1)
    degp = _make_sc_degree()(dst_p).reshape(NC, N_PAD, 1)

    y1, dinv = pl.pallas_call(
        _tc_scale_body,
        out_shape=[
            jax.ShapeDtypeStruct((N_PAD, H), f32),
            jax.ShapeDtypeStruct((N_PAD, 1), f32),
        ],
    )(xw, degp)

    p1 = _make_sc_aggregate()(y1, src_p, dst_p)

    y2 = pl.pallas_call(
        _tc_mid_body,
        out_shape=jax.ShapeDtypeStruct((N_PAD, H), f32),
    )(p1, y1, dinv, W2, b1.reshape(1, H))

    p2 = _make_sc_aggregate()(y2, src_p, dst_p)

    out = pl.pallas_call(
        _tc_head_body,
        out_shape=jax.ShapeDtypeStruct((G, C), f32),
    )(p2, y2, dinv, b2.reshape(1, H), batch_p,
      Wl1, bl1.reshape(1, H), Wl2, bl2.reshape(1, C))
    return out


# speedup vs baseline: 36.3714x; 1.0643x over previous
import functools

import jax
import jax.numpy as jnp
from jax import lax
from jax.experimental import pallas as pl
from jax.experimental.pallas import tpu as pltpu
from jax.experimental.pallas import tpu_sc as plsc

N = 10000
E = 320000
D = 128
H = 128
C = 10
G = 64

NC = 2
NS = 16
NW = NC * NS
LANES = 16

CH = 64
NCHUNK = 160
PH = 4
NBUF = 4
CPP = NCHUNK // PH
EPT = CH * NCHUNK
E_PAD = EPT * NW
N_PAD = 10240
RPT = N_PAD // NS


@functools.cache
def _make_sc_degree():
    mesh = plsc.VectorSubcoreMesh(
        core_axis_name="c", subcore_axis_name="s",
        num_cores=NC, num_subcores=NS,
    )
    return pl.kernel(
        _sc_degree,
        out_type=jax.ShapeDtypeStruct((NC, 1, N_PAD), jnp.float32),
        mesh=mesh,
        scratch_types=[
            pltpu.VMEM((NCHUNK, CH), jnp.int32),
            pltpu.VMEM((CH,), jnp.float32),
            pltpu.VMEM((RPT,), jnp.float32),
            pltpu.VMEM_SHARED((N_PAD,), jnp.float32),
        ],
    )


def _sc_degree(dst_hbm, out_hbm, dstv, ones_v, stage_v, acc_sh):
    c = lax.axis_index("c")
    s = lax.axis_index("s")
    wid = c * NS + s
    pltpu.sync_copy(dst_hbm.at[pl.ds(wid * NCHUNK, NCHUNK)], dstv)
    for j in range(CH // LANES):
        ones_v[pl.ds(j * LANES, LANES)] = jnp.ones((LANES,), jnp.float32)
    zero = jnp.zeros((LANES,), jnp.float32)

    def zfill(i, carry):
        stage_v[pl.ds(i * LANES, LANES)] = zero
        return carry

    lax.fori_loop(0, RPT // LANES, zfill, 0)
    pltpu.sync_copy(stage_v, acc_sh.at[pl.ds(s * RPT, RPT)])
    plsc.subcore_barrier()

    def body(g, carry):
        pltpu.sync_copy(ones_v, acc_sh.at[dstv.at[g]], add=True)
        return carry

    lax.fori_loop(0, NCHUNK, body, 0)
    plsc.subcore_barrier()
    pltpu.sync_copy(acc_sh.at[pl.ds(s * RPT, RPT)], stage_v)
    pltpu.sync_copy(stage_v, out_hbm.at[c, 0, pl.ds(s * RPT, RPT)])


@functools.cache
def _make_sc_aggregate():
    mesh = plsc.VectorSubcoreMesh(
        core_axis_name="c", subcore_axis_name="s",
        num_cores=NC, num_subcores=NS,
    )
    return pl.kernel(
        _sc_aggregate,
        out_type=jax.ShapeDtypeStruct((NC, N_PAD, H), jnp.float32),
        mesh=mesh,
        scratch_types=[
            pltpu.VMEM((EPT,), jnp.int32),
            pltpu.VMEM((CPP, CH), jnp.int32),
            pltpu.VMEM((NBUF, CH, H), jnp.float32),
            pltpu.VMEM_SHARED((N_PAD, H), jnp.float32),
            [pltpu.SemaphoreType.DMA] * NBUF,
        ],
    )


def _sc_aggregate(y_hbm, src_hbm, dst_hbm, out_hbm, srcv, dstv, rows, acc_sh,
                  sems):
    c = lax.axis_index("c")
    s = lax.axis_index("s")
    wid = c * NS + s

    zero = jnp.zeros((LANES,), jnp.float32)

    def zfill(i, carry):
        for j in range(H // LANES):
            rows[0, i, pl.ds(j * LANES, LANES)] = zero
        return carry

    lax.fori_loop(0, CH, zfill, 0)
    for k in range(RPT // CH):
        pltpu.sync_copy(rows.at[0], acc_sh.at[pl.ds(s * RPT + k * CH, CH)])
    plsc.subcore_barrier()

    pltpu.sync_copy(src_hbm.at[pl.ds(wid * EPT, EPT)], srcv)

    def gather(p, g, buf):
        idx = srcv.at[pl.ds((p * CPP + g) * CH, CH)]
        return pltpu.async_copy(y_hbm.at[idx], rows.at[buf], sems[buf])

    def gwait(p, g, buf):
        idx = srcv.at[pl.ds((p * CPP + g) * CH, CH)]
        pltpu.make_async_copy(y_hbm.at[idx], rows.at[buf], sems[buf]).wait()

    for p in range(PH):
        pltpu.sync_copy(
            dst_hbm.at[pl.ds(wid * NCHUNK + p * CPP, CPP)], dstv)
        for b in range(NBUF - 1):
            gather(p, b, b)

        def body(j, carry):
            g0 = NBUF * j
            gather(p, g0 + NBUF - 1, NBUF - 1)
            for b in range(NBUF):
                g = g0 + b
                gwait(p, g, b)
                pltpu.sync_copy(rows.at[b], acc_sh.at[dstv.at[g]], add=True)
                if b < NBUF - 1:
                    @pl.when(g + NBUF < CPP)
                    def _():
                        gather(p, g + NBUF, b)
            return carry

        lax.fori_loop(0, CPP // NBUF, body, 0)
    plsc.subcore_barrier()
    nwr = RPT // CH
    for k in range(nwr):
        b = k % NBUF
        if k >= NBUF:
            ko = k - NBUF
            pltpu.make_async_copy(
                rows.at[b],
                out_hbm.at[c, pl.ds(s * RPT + ko * CH, CH)],
                sems[b],
            ).wait()
        pltpu.sync_copy(acc_sh.at[pl.ds(s * RPT + k * CH, CH)], rows.at[b])
        pltpu.async_copy(
            rows.at[b], out_hbm.at[c, pl.ds(s * RPT + k * CH, CH)], sems[b])
    for k in range(max(nwr - NBUF, 0), nwr):
        b = k % NBUF
        pltpu.make_async_copy(
            rows.at[b],
            out_hbm.at[c, pl.ds(s * RPT + k * CH, CH)],
            sems[b],
        ).wait()


def _tc_mm1_body(x_ref, w_ref, xw_ref):
    xw_ref[0:N, :] = jnp.dot(
        x_ref[...], w_ref[...], preferred_element_type=jnp.float32)
    xw_ref[N:N_PAD, :] = jnp.zeros((N_PAD - N, H), jnp.float32)


def _tc_scale_body(xw_ref, degp_ref, y_ref, dinv_ref):
    deg = degp_ref[0] + degp_ref[1] + 1.0
    dinv = lax.rsqrt(deg)
    y_ref[...] = xw_ref[...] * jnp.transpose(dinv)
    dinv_ref[...] = dinv


def _tc_mid_body(p_ref, y1_ref, dinv_ref, w2_ref, b1_ref, y2_ref):
    dinv = jnp.transpose(dinv_ref[...])
    h1 = (p_ref[0] + p_ref[1] + y1_ref[...]) * dinv + b1_ref[...]
    h1 = jnp.maximum(h1, 0.0)
    y2_ref[...] = (
        jnp.dot(h1, w2_ref[...], preferred_element_type=jnp.float32) * dinv
    )


def _tc_head_body(p_ref, y2_ref, dinv_ref, b2_ref, batch_ref,
                  wl1_ref, bl1_ref, wl2_ref, bl2_ref, out_ref):
    dinv = jnp.transpose(dinv_ref[...])
    h2 = (p_ref[0] + p_ref[1] + y2_ref[...]) * dinv + b2_ref[...]
    h2 = jnp.maximum(h2, 0.0)
    ids = batch_ref[...]
    gidx = lax.broadcasted_iota(jnp.int32, (G, N_PAD), 0)
    oh = (gidx == ids).astype(jnp.float32)
    seg = jnp.dot(oh, h2, preferred_element_type=jnp.float32)
    cnt = jnp.sum(oh, axis=1, keepdims=True)
    pooled = seg / jnp.maximum(cnt, 1.0)
    z = jnp.dot(pooled, wl1_ref[...], preferred_element_type=jnp.float32)
    z = jnp.maximum(z + bl1_ref[...], 0.0)
    logits = jnp.dot(z, wl2_ref[...], preferred_element_type=jnp.float32)
    logits = logits + bl2_ref[...]
    m = jnp.max(logits, axis=1, keepdims=True)
    lse = jnp.log(jnp.sum(jnp.exp(logits - m), axis=1, keepdims=True)) + m
    out_ref[...] = logits - lse


def kernel(x, edge_index, batch, W1, b1, W2, b2, Wl1, bl1, Wl2, bl2):
    f32 = jnp.float32
    src = edge_index[0]
    dst = edge_index[1]

    n_pad_e = E_PAD - E
    dummy = N + (jnp.arange(n_pad_e, dtype=jnp.int32) % (N_PAD - N))
    src_p = jnp.concatenate([src, dummy])
    dst_p = jnp.concatenate([dst, dummy]).reshape(NW * NCHUNK, CH)

    batch_p = jnp.concatenate(
        [batch, jnp.full((N_PAD - N,), G, jnp.int32)]
    ).reshape(1, N_PAD)

    xw = pl.pallas_call(
        _tc_mm1_body,
        out_shape=jax.ShapeDtypeStruct((N_PAD, H), f32),
    )(x, W1)
    degp = _make_sc_degree()(dst_p)

    y1, dinv = pl.pallas_call(
        _tc_scale_body,
        out_shape=[
            jax.ShapeDtypeStruct((N_PAD, H), f32),
            jax.ShapeDtypeStruct((1, N_PAD), f32),
        ],
    )(xw, degp)

    p1 = _make_sc_aggregate()(y1, src_p, dst_p)

    y2 = pl.pallas_call(
        _tc_mid_body,
        out_shape=jax.ShapeDtypeStruct((N_PAD, H), f32),
    )(p1, y1, dinv, W2, b1.reshape(1, H))

    p2 = _make_sc_aggregate()(y2, src_p, dst_p)

    out = pl.pallas_call(
        _tc_head_body,
        out_shape=jax.ShapeDtypeStruct((G, C), f32),
    )(p2, y2, dinv, b2.reshape(1, H), batch_p,
      Wl1, bl1.reshape(1, H), Wl2, bl2.reshape(1, C))
    return out
